# baseline (speedup 1.0000x reference)
import dataclasses
import functools

import jax
import jax.numpy as jnp
from jax import lax
from jax.experimental import pallas as pl
from jax.experimental.pallas import tpu as pltpu
from jax.experimental.pallas import tpu_sc as plsc

_B, _S, _K, _C = 4, 2048, 1024, 42
_R = 2 * _B
_L = 16
_W = _S // _L
_CHUNKS = _W // _L


def _sc_gather(bert2d, e1w, e2w):
    mesh = plsc.VectorSubcoreMesh(core_axis_name="c", subcore_axis_name="s")
    cp = pltpu.CompilerParams()
    if "needs_layout_passes" in pltpu.CompilerParams.__dataclass_fields__:
        cp = dataclasses.replace(cp, needs_layout_passes=False)

    @functools.partial(
        pl.kernel,
        mesh=mesh,
        compiler_params=cp,
        out_type=jax.ShapeDtypeStruct((_B, 2 * _K), jnp.float32),
        scratch_types=[
            pltpu.VMEM((1, _W), jnp.int32),
            pltpu.VMEM((1, _K), jnp.float32),
        ],
    )
    def k(bert_hbm, e1_hbm, e2_hbm, out_hbm, mask_v, row_v):
        core = lax.axis_index("c")
        b_idx = lax.axis_index("s")

        def gather_one(m_hbm, ent):
            pltpu.sync_copy(m_hbm.at[pl.ds(b_idx, 1), :], mask_v)
            base = lax.iota(jnp.int32, _L)

            def body(i, acc):
                v = mask_v[0, pl.ds(i * _L, _L)]
                g = base + i * _L
                cand = jnp.where(v != 0, _L * g + v - 1, 0)
                return jnp.maximum(acc, cand)

            acc = lax.fori_loop(0, _CHUNKS, body, jnp.zeros((_L,), jnp.int32))
            pos = jnp.max(acc, axis=0)
            pltpu.sync_copy(bert_hbm.at[pl.ds(b_idx * _S + pos, 1), :], row_v)
            pltpu.sync_copy(
                row_v, out_hbm.at[pl.ds(b_idx, 1), pl.ds(ent * _K, _K)]
            )

        @pl.when((core == 0) & (b_idx < _B))
        def _():
            gather_one(e1_hbm, 0)

        @pl.when((core == 1) & (b_idx < _B))
        def _():
            gather_one(e2_hbm, 1)

    return k(bert2d, e1w, e2w)


def _tc_head(dense, W, b2):
    def body(x_ref, w_ref, b_ref, o_ref):
        logits = (
            jnp.dot(x_ref[...], w_ref[...], preferred_element_type=jnp.float32)
            + b_ref[...]
        )
        m = jnp.max(logits, axis=-1, keepdims=True)
        e = jnp.exp(logits - m)
        o_ref[...] = e / jnp.sum(e, axis=-1, keepdims=True)

    return pl.pallas_call(
        body,
        out_shape=jax.ShapeDtypeStruct((_B, _C), jnp.float32),
    )(dense, W, b2)


def _pack_words(mask):
    codes = (jnp.arange(_S, dtype=jnp.int32) % _L + 1)[None, :]
    vals = mask.astype(jnp.int32) * codes
    return lax.reduce_window(
        vals, 0, lax.max, (1, _L), (1, _L), "VALID"
    )


def kernel(bert_output, e1_mask, e2_mask, W, b):
    bert2d = bert_output.reshape(_B * _S, _K)
    dense = _sc_gather(bert2d, _pack_words(e1_mask), _pack_words(e2_mask))
    out = _tc_head(dense, W, b.reshape(1, _C))
    return out.reshape(_B, 1, _C)


# speedup vs baseline: 1.0445x; 1.0445x over previous
import dataclasses
import functools

import jax
import jax.numpy as jnp
from jax import lax
from jax.experimental import pallas as pl
from jax.experimental.pallas import tpu as pltpu
from jax.experimental.pallas import tpu_sc as plsc

_B, _S, _K, _C = 4, 2048, 1024, 42
_R = 2 * _B
_L = 16
_W = _S // _L
_CHUNKS = _W // _L


def _sc_gather(bert2d, masks_w):
    mesh = plsc.VectorSubcoreMesh(core_axis_name="c", subcore_axis_name="s")
    cp = pltpu.CompilerParams()
    if "needs_layout_passes" in pltpu.CompilerParams.__dataclass_fields__:
        cp = dataclasses.replace(cp, needs_layout_passes=False)

    @functools.partial(
        pl.kernel,
        mesh=mesh,
        compiler_params=cp,
        out_type=jax.ShapeDtypeStruct((_B, 2 * _K), jnp.float32),
        scratch_types=[
            pltpu.VMEM((_W,), jnp.int32),
            pltpu.VMEM((1, _K), jnp.float32),
        ],
    )
    def k(bert_hbm, masks_hbm, out_hbm, mask_v, row_v):
        w = lax.axis_index("s") * 2 + lax.axis_index("c")

        @pl.when(w < _R)
        def _():
            pltpu.sync_copy(masks_hbm.at[pl.ds(w * _W, _W)], mask_v)
            base = lax.iota(jnp.int32, _L)

            def body(i, acc):
                v = mask_v[pl.ds(i * _L, _L)]
                g = base + i * _L
                cand = jnp.where(v != 0, _L * g + v - 1, 0)
                return jnp.maximum(acc, cand)

            acc = lax.fori_loop(0, _CHUNKS, body, jnp.zeros((_L,), jnp.int32))
            pos = jnp.max(acc, axis=0)
            b_idx = w // 2
            e_idx = w % 2
            pltpu.sync_copy(bert_hbm.at[pl.ds(b_idx * _S + pos, 1), :], row_v)
            pltpu.sync_copy(
                row_v, out_hbm.at[pl.ds(b_idx, 1), pl.ds(e_idx * _K, _K)]
            )

    return k(bert2d, masks_w)


def _tc_head(dense, W, b2):
    def body(x_ref, w_ref, b_ref, o_ref):
        logits = (
            jnp.dot(x_ref[...], w_ref[...], preferred_element_type=jnp.float32)
            + b_ref[...]
        )
        m = jnp.max(logits, axis=-1, keepdims=True)
        e = jnp.exp(logits - m)
        o_ref[...] = e / jnp.sum(e, axis=-1, keepdims=True)

    return pl.pallas_call(
        body,
        out_shape=jax.ShapeDtypeStruct((_B, _C), jnp.float32),
    )(dense, W, b2)


def _pack_words(e1_mask, e2_mask):
    m = jnp.stack([e1_mask, e2_mask], axis=1).reshape(_R, _W, _L)
    codes = (jnp.arange(_L, dtype=jnp.int32) + 1)[None, None, :]
    return (m.astype(jnp.int32) * codes).sum(axis=-1).reshape(_R * _W)


def kernel(bert_output, e1_mask, e2_mask, W, b):
    bert2d = bert_output.reshape(_B * _S, _K)
    dense = _sc_gather(bert2d, _pack_words(e1_mask, e2_mask))
    out = _tc_head(dense, W, b.reshape(1, _C))
    return out.reshape(_B, 1, _C)
